# baseline (speedup 1.0000x reference)
import jax
import jax.numpy as jnp
from jax.experimental import pallas as pl
from jax.experimental.pallas import tpu as pltpu

_ROWS = 1_000_000
_DIM = 64
_NCHUNK = 8
_CHUNK_ROWS = _ROWS // _NCHUNK


def _copy_body(in_ref, out_ref, *sems):
    copies = []
    for c in range(_NCHUNK):
        sl = pl.ds(c * _CHUNK_ROWS, _CHUNK_ROWS)
        copies.append(
            pltpu.make_async_copy(in_ref.at[sl, :], out_ref.at[sl, :], sems[c])
        )
    for cp in copies:
        cp.start()
    for cp in copies:
        cp.wait()


def kernel(path_emb):
    return pl.pallas_call(
        _copy_body,
        in_specs=[pl.BlockSpec(memory_space=pl.ANY)],
        out_specs=pl.BlockSpec(memory_space=pl.ANY),
        out_shape=jax.ShapeDtypeStruct((_ROWS, _DIM), jnp.float32),
        scratch_shapes=[pltpu.SemaphoreType.DMA] * _NCHUNK,
    )(path_emb)


# speedup vs baseline: 16.1441x; 16.1441x over previous
import jax
import jax.numpy as jnp
from jax.experimental import pallas as pl

_ROWS = 1_000_000
_DIM = 64
_BLOCK_ROWS = 8_000


def _copy_block(in_ref, out_ref):
    out_ref[...] = in_ref[...]


def kernel(path_emb):
    return pl.pallas_call(
        _copy_block,
        grid=(_ROWS // _BLOCK_ROWS,),
        in_specs=[pl.BlockSpec((_BLOCK_ROWS, _DIM), lambda i: (i, 0))],
        out_specs=pl.BlockSpec((_BLOCK_ROWS, _DIM), lambda i: (i, 0)),
        out_shape=jax.ShapeDtypeStruct((_ROWS, _DIM), jnp.float32),
    )(path_emb)
